# baseline (speedup 1.0000x reference)
import functools

import jax
import jax.numpy as jnp
from jax import lax
from jax.experimental import pallas as pl
from jax.experimental.pallas import tpu as pltpu
from jax.experimental.pallas import tpu_sc as plsc

B = 16384
D = 32
NC = 2
NS = 16
NW = NC * NS
BPW = B // NW
L = 16

_sc_mesh = plsc.VectorSubcoreMesh(core_axis_name="c", subcore_axis_name="s")


@functools.partial(
    pl.kernel,
    mesh=_sc_mesh,
    compiler_params=pltpu.CompilerParams(needs_layout_passes=False),
    out_type=[jax.ShapeDtypeStruct((D, B), jnp.float32) for _ in range(4)],
    scratch_types=[
        pltpu.VMEM((BPW,), jnp.int32),
        pltpu.VMEM((BPW,), jnp.int32),
        pltpu.VMEM((2, 4, D, 128), jnp.float32),
        pltpu.VMEM((D, BPW), jnp.float32),
        pltpu.VMEM((D, BPW), jnp.float32),
        pltpu.VMEM((D, BPW), jnp.float32),
        pltpu.VMEM((D, BPW), jnp.float32),
        pltpu.SemaphoreType.DMA,
        pltpu.SemaphoreType.DMA,
    ],
)
def _sc_gather(user_hbm, item_hbm, ugt, igt, umt, imt,
               ug_o, ig_o, um_o, im_o,
               uidx, iidx, slabs, st0, st1, st2, st3, sem0, sem1):
    wid = lax.axis_index("s") * NC + lax.axis_index("c")
    base = wid * BPW
    pltpu.sync_copy(user_hbm.at[pl.ds(base, BPW)], uidx)
    pltpu.sync_copy(item_hbm.at[pl.ds(base, BPW)], iidx)

    tabs = (ugt, umt, igt, imt)
    stages = (st0, st2, st1, st3)
    sems = (sem0, sem1)
    row_lo = lax.broadcasted_iota(jnp.int32, (L,), 0)
    row_hi = row_lo + L

    def slab_off(r):
        return pl.multiple_of((r // 128) * 128, 128)

    def fire(ru, ri, slot):
        sem = sems[slot]
        ou, oi = slab_off(ru), slab_off(ri)
        offs = (ou, ou, oi, oi)
        for t in range(4):
            for cg in range(4):
                pltpu.async_copy(
                    tabs[t].at[pl.ds(cg * 8, 8), pl.ds(offs[t], 128)],
                    slabs.at[slot, t, pl.ds(cg * 8, 8)], sem)

    def wait(slot):
        sem = sems[slot]
        for t in range(4):
            for cg in range(4):
                pltpu.make_async_copy(
                    tabs[t].at[pl.ds(cg * 8, 8), pl.ds(0, 128)],
                    slabs.at[slot, t, pl.ds(cg * 8, 8)], sem).wait()

    def extract(j, ru, ri, slot):
        js = jnp.full((L,), j, jnp.int32)
        lanes = (ru % 128, ru % 128, ri % 128, ri % 128)
        for t in range(4):
            lv = jnp.full((L,), lanes[t], jnp.int32)
            v0 = plsc.load_gather(slabs.at[slot, t], [row_lo, lv])
            v1 = plsc.load_gather(slabs.at[slot, t], [row_hi, lv])
            plsc.store_scatter(stages[t], [row_lo, js], v0)
            plsc.store_scatter(stages[t], [row_hi, js], v1)

    uv0 = uidx[pl.ds(0, L)]
    iv0 = iidx[pl.ds(0, L)]
    fire(uv0[0], iv0[0], 0)

    def group_body(g, carry):
        g_nxt = jnp.minimum(g + 1, BPW // L - 1)
        uv = uidx[pl.ds(g * L, L)]
        iv = iidx[pl.ds(g * L, L)]
        uvn = uidx[pl.ds(g_nxt * L, L)]
        ivn = iidx[pl.ds(g_nxt * L, L)]
        for lane in range(L):
            j = g * L + lane
            slot = lane & 1
            if lane < L - 1:
                run, rin = uv[lane + 1], iv[lane + 1]
            else:
                run, rin = uvn[0], ivn[0]
            fire(run, rin, 1 - slot)
            wait(slot)
            extract(j, uv[lane], iv[lane], slot)
        return carry

    lax.fori_loop(0, BPW // L, group_body, 0)
    wait(0)

    pltpu.sync_copy(st0, ug_o.at[:, pl.ds(base, BPW)])
    pltpu.sync_copy(st1, ig_o.at[:, pl.ds(base, BPW)])
    pltpu.sync_copy(st2, um_o.at[:, pl.ds(base, BPW)])
    pltpu.sync_copy(st3, im_o.at[:, pl.ds(base, BPW)])


_BLK = 2048


def _mlp_body(ug, ig, um, im, w1a, w1b, b1, w2, b2, wpg, wph, bp, out):
    gmf = ug[...] * ig[...]
    h = (jnp.dot(w1a[...], um[...], preferred_element_type=jnp.float32)
         + jnp.dot(w1b[...], im[...], preferred_element_type=jnp.float32)
         + b1[...])
    h = jnp.maximum(h, 0.0)
    h = jnp.dot(w2[...], h, preferred_element_type=jnp.float32) + b2[...]
    h = jnp.maximum(h, 0.0)
    y = (jnp.dot(wpg[...], gmf[...], preferred_element_type=jnp.float32)
         + jnp.dot(wph[...], h[...], preferred_element_type=jnp.float32)
         + bp[...])
    out[...] = y


_col_spec = pl.BlockSpec((D, _BLK), lambda i: (0, i))


def _full(shape):
    return pl.BlockSpec(shape, lambda i: tuple(0 for _ in shape))


_mlp_call = pl.pallas_call(
    _mlp_body,
    grid=(B // _BLK,),
    in_specs=[
        _col_spec, _col_spec, _col_spec, _col_spec,
        _full((D, D)), _full((D, D)), _full((D, 1)),
        _full((16, D)), _full((16, 1)),
        _full((1, D)), _full((1, 16)), _full((1, 1)),
    ],
    out_specs=pl.BlockSpec((1, _BLK), lambda i: (0, i)),
    out_shape=jax.ShapeDtypeStruct((1, B), jnp.float32),
)


def kernel(user, item, user_gmf, item_gmf, user_mlp, item_mlp,
           W1, b1, W2, b2, Wp, bp):
    user = user.astype(jnp.int32)
    item = item.astype(jnp.int32)
    ug, ig, um, im = _sc_gather(user, item,
                                user_gmf.T, item_gmf.T,
                                user_mlp.T, item_mlp.T)
    y = _mlp_call(ug, ig, um, im,
                  W1[:D].T, W1[D:].T, b1.reshape(D, 1),
                  W2.T, b2.reshape(16, 1),
                  Wp[:D].T, Wp[D:].T, bp.reshape(1, 1))
    return y.reshape(B)


# speedup vs baseline: 1.1171x; 1.1171x over previous
import functools

import jax
import jax.numpy as jnp
from jax import lax
from jax.experimental import pallas as pl
from jax.experimental.pallas import tpu as pltpu
from jax.experimental.pallas import tpu_sc as plsc

B = 16384
D = 32
NC = 2
NS = 16
NW = NC * NS
BPW = B // NW
L = 16

_sc_mesh = plsc.VectorSubcoreMesh(core_axis_name="c", subcore_axis_name="s")


@functools.partial(
    pl.kernel,
    mesh=_sc_mesh,
    compiler_params=pltpu.CompilerParams(needs_layout_passes=False),
    out_type=[jax.ShapeDtypeStruct((D, B), jnp.float32) for _ in range(4)],
    scratch_types=[
        pltpu.VMEM((BPW,), jnp.int32),
        pltpu.VMEM((BPW,), jnp.int32),
        pltpu.VMEM((3, 4, D, 128), jnp.float32),
        pltpu.VMEM((D, BPW), jnp.float32),
        pltpu.VMEM((D, BPW), jnp.float32),
        pltpu.VMEM((D, BPW), jnp.float32),
        pltpu.VMEM((D, BPW), jnp.float32),
        pltpu.SemaphoreType.DMA,
        pltpu.SemaphoreType.DMA,
        pltpu.SemaphoreType.DMA,
    ],
)
def _sc_gather(user_hbm, item_hbm, ugt, igt, umt, imt,
               ug_o, ig_o, um_o, im_o,
               uidx, iidx, slabs, st0, st1, st2, st3, sem0, sem1, sem2):
    wid = lax.axis_index("s") * NC + lax.axis_index("c")
    base = wid * BPW
    pltpu.sync_copy(user_hbm.at[pl.ds(base, BPW)], uidx)
    pltpu.sync_copy(item_hbm.at[pl.ds(base, BPW)], iidx)

    tabs = (ugt, umt, igt, imt)
    stages = (st0, st2, st1, st3)
    sems = (sem0, sem1, sem2)
    row_lo = lax.broadcasted_iota(jnp.int32, (L,), 0)
    row_hi = row_lo + L

    def slab_off(r):
        return pl.multiple_of((r // 128) * 128, 128)

    def fire(ru, ri, slot):
        sem = sems[slot]
        ou, oi = slab_off(ru), slab_off(ri)
        offs = (ou, ou, oi, oi)
        for t in range(4):
            pltpu.async_copy(tabs[t].at[:, pl.ds(offs[t], 128)],
                             slabs.at[slot, t], sem)

    def wait(slot):
        sem = sems[slot]
        for t in range(4):
            pltpu.make_async_copy(tabs[t].at[:, pl.ds(0, 128)],
                                  slabs.at[slot, t], sem).wait()

    def extract(j, ru, ri, slot):
        js = jnp.full((L,), j, jnp.int32)
        lanes = (ru % 128, ru % 128, ri % 128, ri % 128)
        for t in range(4):
            lv = jnp.full((L,), lanes[t], jnp.int32)
            v0 = plsc.load_gather(slabs.at[slot, t], [row_lo, lv])
            v1 = plsc.load_gather(slabs.at[slot, t], [row_hi, lv])
            plsc.store_scatter(stages[t], [row_lo, js], v0)
            plsc.store_scatter(stages[t], [row_hi, js], v1)

    BLK48 = 3 * L
    NBLK = 480 // BLK48

    def vecs(base, m):
        return (uidx[pl.ds(base + m * L, L)], iidx[pl.ds(base + m * L, L)])

    uv0, iv0 = vecs(0, 0)
    fire(uv0[0], iv0[0], 0)
    fire(uv0[1], iv0[1], 1)

    def block_body(b, carry):
        base = b * BLK48
        uvs, ivs = zip(*(vecs(base, m) for m in range(4)))
        for k in range(BLK48):
            j = base + k
            slot = k % 3
            m2, l2 = (k + 2) // L, (k + 2) % L
            fire(uvs[m2][l2], ivs[m2][l2], (k + 2) % 3)
            wait(slot)
            extract(j, uvs[k // L][k % L], ivs[k // L][k % L], slot)
        return carry

    lax.fori_loop(0, NBLK, block_body, 0)

    tbase = NBLK * BLK48
    uvs, ivs = zip(*(vecs(tbase, m) for m in range(2)))
    for k in range(BPW - tbase):
        j = tbase + k
        slot = k % 3
        kf = min(k + 2, BPW - tbase - 1)
        m2, l2 = kf // L, kf % L
        fire(uvs[m2][l2], ivs[m2][l2], (k + 2) % 3)
        wait(slot)
        extract(j, uvs[k // L][k % L], ivs[k // L][k % L], slot)
    wait(2)
    wait(0)

    pltpu.sync_copy(st0, ug_o.at[:, pl.ds(base, BPW)])
    pltpu.sync_copy(st1, ig_o.at[:, pl.ds(base, BPW)])
    pltpu.sync_copy(st2, um_o.at[:, pl.ds(base, BPW)])
    pltpu.sync_copy(st3, im_o.at[:, pl.ds(base, BPW)])


_BLK = 2048


def _mlp_body(ug, ig, um, im, w1a, w1b, b1, w2, b2, wpg, wph, bp, out):
    gmf = ug[...] * ig[...]
    h = (jnp.dot(w1a[...], um[...], preferred_element_type=jnp.float32)
         + jnp.dot(w1b[...], im[...], preferred_element_type=jnp.float32)
         + b1[...])
    h = jnp.maximum(h, 0.0)
    h = jnp.dot(w2[...], h, preferred_element_type=jnp.float32) + b2[...]
    h = jnp.maximum(h, 0.0)
    y = (jnp.dot(wpg[...], gmf[...], preferred_element_type=jnp.float32)
         + jnp.dot(wph[...], h[...], preferred_element_type=jnp.float32)
         + bp[...])
    out[...] = y


_col_spec = pl.BlockSpec((D, _BLK), lambda i: (0, i))


def _full(shape):
    return pl.BlockSpec(shape, lambda i: tuple(0 for _ in shape))


_mlp_call = pl.pallas_call(
    _mlp_body,
    grid=(B // _BLK,),
    in_specs=[
        _col_spec, _col_spec, _col_spec, _col_spec,
        _full((D, D)), _full((D, D)), _full((D, 1)),
        _full((16, D)), _full((16, 1)),
        _full((1, D)), _full((1, 16)), _full((1, 1)),
    ],
    out_specs=pl.BlockSpec((1, _BLK), lambda i: (0, i)),
    out_shape=jax.ShapeDtypeStruct((1, B), jnp.float32),
)


def kernel(user, item, user_gmf, item_gmf, user_mlp, item_mlp,
           W1, b1, W2, b2, Wp, bp):
    user = user.astype(jnp.int32)
    item = item.astype(jnp.int32)
    ug, ig, um, im = _sc_gather(user, item,
                                user_gmf.T, item_gmf.T,
                                user_mlp.T, item_mlp.T)
    y = _mlp_call(ug, ig, um, im,
                  W1[:D].T, W1[D:].T, b1.reshape(D, 1),
                  W2.T, b2.reshape(16, 1),
                  Wp[:D].T, Wp[D:].T, bp.reshape(1, 1))
    return y.reshape(B)


# speedup vs baseline: 1.2232x; 1.0950x over previous
import functools

import jax
import jax.numpy as jnp
from jax import lax
from jax.experimental import pallas as pl
from jax.experimental.pallas import tpu as pltpu
from jax.experimental.pallas import tpu_sc as plsc

B = 16384
D = 32
NC = 2
NS = 16
NW = NC * NS
BPW = B // NW
L = 16

_sc_mesh = plsc.VectorSubcoreMesh(core_axis_name="c", subcore_axis_name="s")


@functools.partial(
    pl.kernel,
    mesh=_sc_mesh,
    compiler_params=pltpu.CompilerParams(needs_layout_passes=False),
    out_type=[jax.ShapeDtypeStruct((D, B), jnp.float32) for _ in range(4)],
    scratch_types=[
        pltpu.VMEM((BPW,), jnp.int32),
        pltpu.VMEM((BPW,), jnp.int32),
        pltpu.VMEM((4, 4, D, 128), jnp.float32),
        pltpu.VMEM((D, BPW // 2), jnp.float32),
        pltpu.VMEM((D, BPW // 2), jnp.float32),
        pltpu.VMEM((D, BPW // 2), jnp.float32),
        pltpu.VMEM((D, BPW // 2), jnp.float32),
        pltpu.SemaphoreType.DMA,
        pltpu.SemaphoreType.DMA,
        pltpu.SemaphoreType.DMA,
        pltpu.SemaphoreType.DMA,
    ],
)
def _sc_gather(user_hbm, item_hbm, ugt, igt, umt, imt,
               ug_o, ig_o, um_o, im_o,
               uidx, iidx, slabs, st0, st1, st2, st3,
               sem0, sem1, sem2, sem3):
    wid = lax.axis_index("s") * NC + lax.axis_index("c")
    base = wid * BPW
    pltpu.sync_copy(user_hbm.at[pl.ds(base, BPW)], uidx)
    pltpu.sync_copy(item_hbm.at[pl.ds(base, BPW)], iidx)

    tabs = (ugt, umt, igt, imt)
    stages = (st0, st2, st1, st3)
    outs = (ug_o, um_o, ig_o, im_o)
    sems = (sem0, sem1, sem2, sem3)
    row_lo = lax.broadcasted_iota(jnp.int32, (L,), 0)
    row_hi = row_lo + L
    H = BPW // 2

    def slab_off(r):
        return pl.multiple_of((r // 128) * 128, 128)

    def fire(ru, ri, slot):
        sem = sems[slot]
        ou, oi = slab_off(ru), slab_off(ri)
        offs = (ou, ou, oi, oi)
        for t in range(4):
            pltpu.async_copy(tabs[t].at[:, pl.ds(offs[t], 128)],
                             slabs.at[slot, t], sem)

    def wait(slot):
        sem = sems[slot]
        for t in range(4):
            pltpu.make_async_copy(tabs[t].at[:, pl.ds(0, 128)],
                                  slabs.at[slot, t], sem).wait()

    def extract(j_local, ru, ri, slot):
        js = jnp.full((L,), j_local, jnp.int32)
        lanes = (ru % 128, ru % 128, ri % 128, ri % 128)
        for t in range(4):
            lv = jnp.full((L,), lanes[t], jnp.int32)
            v0 = plsc.load_gather(slabs.at[slot, t], [row_lo, lv])
            v1 = plsc.load_gather(slabs.at[slot, t], [row_hi, lv])
            plsc.store_scatter(stages[t], [row_lo, js], v0)
            plsc.store_scatter(stages[t], [row_hi, js], v1)

    def vecs(base, m):
        return (uidx[pl.ds(base + m * L, L)], iidx[pl.ds(base + m * L, L)])

    uv0, iv0 = vecs(0, 0)
    for p in range(3):
        fire(uv0[p], iv0[p], p)

    def make_group_body(half_off):
        def group_body(g, carry):
            g_nxt = jnp.minimum(g + 1, BPW // L - 1)
            uv = uidx[pl.ds(g * L, L)]
            iv = iidx[pl.ds(g * L, L)]
            uvn = uidx[pl.ds(g_nxt * L, L)]
            ivn = iidx[pl.ds(g_nxt * L, L)]
            for lane in range(L):
                j = g * L + lane
                slot = lane % 4
                lf = lane + 3
                if lf < L:
                    run, rin = uv[lf], iv[lf]
                else:
                    run, rin = uvn[lf - L], ivn[lf - L]
                fire(run, rin, lf % 4)
                wait(slot)
                extract(j - half_off, uv[lane], iv[lane], slot)
            return carry
        return group_body

    NG = BPW // L
    lax.fori_loop(0, NG // 2, make_group_body(0), 0)
    for t in range(4):
        pltpu.sync_copy(stages[t], outs[t].at[:, pl.ds(base, H)])
    lax.fori_loop(NG // 2, NG, make_group_body(H), 0)
    wait(0)
    wait(1)
    wait(2)
    for t in range(4):
        pltpu.sync_copy(stages[t], outs[t].at[:, pl.ds(base + H, H)])


_BLK = 2048


def _mlp_body(ug, ig, um, im, w1a, w1b, b1, w2, b2, wpg, wph, bp, out):
    gmf = ug[...] * ig[...]
    h = (jnp.dot(w1a[...], um[...], preferred_element_type=jnp.float32)
         + jnp.dot(w1b[...], im[...], preferred_element_type=jnp.float32)
         + b1[...])
    h = jnp.maximum(h, 0.0)
    h = jnp.dot(w2[...], h, preferred_element_type=jnp.float32) + b2[...]
    h = jnp.maximum(h, 0.0)
    y = (jnp.dot(wpg[...], gmf[...], preferred_element_type=jnp.float32)
         + jnp.dot(wph[...], h[...], preferred_element_type=jnp.float32)
         + bp[...])
    out[...] = y


_col_spec = pl.BlockSpec((D, _BLK), lambda i: (0, i))


def _full(shape):
    return pl.BlockSpec(shape, lambda i: tuple(0 for _ in shape))


_mlp_call = pl.pallas_call(
    _mlp_body,
    grid=(B // _BLK,),
    in_specs=[
        _col_spec, _col_spec, _col_spec, _col_spec,
        _full((D, D)), _full((D, D)), _full((D, 1)),
        _full((16, D)), _full((16, 1)),
        _full((1, D)), _full((1, 16)), _full((1, 1)),
    ],
    out_specs=pl.BlockSpec((1, _BLK), lambda i: (0, i)),
    out_shape=jax.ShapeDtypeStruct((1, B), jnp.float32),
)


def kernel(user, item, user_gmf, item_gmf, user_mlp, item_mlp,
           W1, b1, W2, b2, Wp, bp):
    user = user.astype(jnp.int32)
    item = item.astype(jnp.int32)
    ug, ig, um, im = _sc_gather(user, item,
                                user_gmf.T, item_gmf.T,
                                user_mlp.T, item_mlp.T)
    y = _mlp_call(ug, ig, um, im,
                  W1[:D].T, W1[D:].T, b1.reshape(D, 1),
                  W2.T, b2.reshape(16, 1),
                  Wp[:D].T, Wp[D:].T, bp.reshape(1, 1))
    return y.reshape(B)
